# baseline (speedup 1.0000x reference)
import functools
import math

import jax
import jax.numpy as jnp
from jax import lax
from jax.experimental import pallas as pl
from jax.experimental.pallas import tpu as pltpu
from jax.experimental.pallas import tpu_sc as plsc

N = 25000
D = 128
C = 64
H = 2
DH = 32
E = 400000

NC = 2
NS = 16
NW = NC * NS
B = 128
NCHUNK = E // B
NP = 25088
ROWS_PER = NP // NS
AW = 48
NLOCP = 2 * ((NCHUNK + 2 * NS - 1) // (2 * NS))
JUNK = NP - 8

R = 1000
GRID = N // R


def _pre_body(x_ref, win_ref, lnw_ref, wcat_ref, bcat_ref,
              xs_ref, q_ref, k_ref, v_ref):
    h = jnp.dot(x_ref[...], win_ref[...], preferred_element_type=jnp.float32)
    h = h + lnw_ref[0:1, :]
    m = jnp.mean(h, axis=-1, keepdims=True)
    v = jnp.mean((h - m) * (h - m), axis=-1, keepdims=True)
    hn = (h - m) / jnp.sqrt(v + 1e-5) * lnw_ref[1:2, :] + lnw_ref[2:3, :]
    xs = jnp.maximum(hn, 0.0)
    qkv = jnp.dot(xs, wcat_ref[0], preferred_element_type=jnp.float32)
    qkv = qkv + bcat_ref[0]
    xs_ref[...] = xs
    q_ref[...] = qkv[:, 0:DH]
    k_ref[...] = qkv[:, DH:2 * DH]
    v_ref[...] = qkv[:, 2 * DH:3 * DH]


_pre_call = pl.pallas_call(
    _pre_body,
    grid=(H, GRID),
    in_specs=[
        pl.BlockSpec((R, D), lambda h, i: (i, 0)),
        pl.BlockSpec((D, C), lambda h, i: (0, 0)),
        pl.BlockSpec((3, C), lambda h, i: (0, 0)),
        pl.BlockSpec((1, C, 3 * DH), lambda h, i: (h, 0, 0)),
        pl.BlockSpec((1, 1, 3 * DH), lambda h, i: (h, 0, 0)),
    ],
    out_specs=[pl.BlockSpec((R, C), lambda h, i: (i, 0))]
    + [pl.BlockSpec((R, DH), lambda h, i: (h * GRID + i, 0))] * 3,
    out_shape=[jax.ShapeDtypeStruct((N, C), jnp.float32)]
    + [jax.ShapeDtypeStruct((H * N, DH), jnp.float32)] * 3,
)


def _post_body(acc_ref, xs_ref, wo_ref, misc_ref, y_ref):
    m0 = acc_ref[0, :, 0:DH]
    m1 = acc_ref[1, :, 0:DH]
    d0 = acc_ref[0, :, DH:DH + 1]
    d1 = acc_ref[1, :, DH:DH + 1]
    msg = jnp.concatenate([m0 / (d0 + 1e-16), m1 / (d1 + 1e-16)], axis=1)
    o = jnp.dot(jax.nn.gelu(msg), wo_ref[...],
                preferred_element_type=jnp.float32) + misc_ref[0:1, :]
    xs = xs_ref[...]
    sig = misc_ref[1:2, :]
    out = sig * o + (1.0 - sig) * xs
    t = out + xs
    m = jnp.mean(t, axis=-1, keepdims=True)
    v = jnp.mean((t - m) * (t - m), axis=-1, keepdims=True)
    y_ref[...] = (t - m) / jnp.sqrt(v + 1e-5) * misc_ref[2:3, :] + misc_ref[3:4, :]


_post_call = pl.pallas_call(
    _post_body,
    grid=(GRID,),
    in_specs=[
        pl.BlockSpec((2, R, AW), lambda i: (0, i, 0)),
        pl.BlockSpec((R, C), lambda i: (i, 0)),
        pl.BlockSpec((C, C), lambda i: (0, 0)),
        pl.BlockSpec((4, C), lambda i: (0, 0)),
    ],
    out_specs=pl.BlockSpec((R, C), lambda i: (i, 0)),
    out_shape=jax.ShapeDtypeStruct((N, C), jnp.float32),
)


def _edge_body(qdA, ksA, vsA, srcA, dstA, qdB, ksB, vsB, srcB, dstB,
               z_hbm, outA, outB,
               isr0, isr1, idr0, idr1, ig0, ig1, is0, is1, sc0, sc1,
               qr0, qr1, kr0, kr1, vr0, vr1, wr0, wr1, acc,
               si0, si1, sq0, sq1, sk0, sk1, sv0, sv1, ss0, ss1):
    isr = (isr0, isr1)
    idr = (idr0, idr1)
    ig = (ig0, ig1)
    isx = (is0, is1)
    sc = (sc0, sc1)
    qr = (qr0, qr1)
    kr = (kr0, kr1)
    vr = (vr0, vr1)
    wr = (wr0, wr1)
    si = (si0, si1)
    sq = (sq0, sq1)
    sk = (sk0, sk1)
    sv = (sv0, sv1)
    ss = (ss0, ss1)

    cid = lax.axis_index("c")
    sid = lax.axis_index("s")
    row0 = sid * ROWS_PER

    offv = jnp.broadcast_to(cid * N, (16,))
    lanes = lax.iota(jnp.int32, 16)
    zvec = jnp.zeros((16,), jnp.float32)
    junkv = jnp.full((16,), JUNK, jnp.int32)

    def run_phase(qd_hbm, ks_hbm, vs_hbm, src_hbm, dst_hbm, out_hbm):
        def base_of(m):
            return jnp.minimum(sid + m * NS, NCHUNK - 1) * B

        def idx_start(m, b):
            ba = base_of(m)
            pltpu.async_copy(src_hbm.at[pl.ds(ba, B)], isr[b], si[b])
            pltpu.async_copy(dst_hbm.at[pl.ds(ba, B)], idr[b], si[b])

        def idx_wait(b):
            pltpu.make_async_copy(src_hbm.at[pl.ds(0, B)], isr[b], si[b]).wait()
            pltpu.make_async_copy(dst_hbm.at[pl.ds(0, B)], idr[b], si[b]).wait()

        def addoff(b):
            for j in range(B // 16):
                sl = pl.ds(j * 16, 16)
                ig[b][sl] = idr[b][sl] + offv
                isx[b][sl] = isr[b][sl] + offv

        def gather_start(b):
            pltpu.async_copy(qd_hbm.at[ig[b]], qr[b], sq[b])
            pltpu.async_copy(ks_hbm.at[isx[b]], kr[b], sk[b])
            pltpu.async_copy(vs_hbm.at[isx[b]], vr[b], sv[b])

        def gather_wait(b):
            pltpu.make_async_copy(qd_hbm.at[ig[b]], qr[b], sq[b]).wait()
            pltpu.make_async_copy(ks_hbm.at[isx[b]], kr[b], sk[b]).wait()
            pltpu.make_async_copy(vs_hbm.at[isx[b]], vr[b], sv[b]).wait()

        def scatter_wait(b):
            pltpu.make_async_copy(wr[b], acc.at[sc[b]], ss[b]).wait()

        idx_start(0, 0)
        idx_wait(0)
        addoff(0)
        gather_start(0)
        idx_start(1, 1)
        plsc.subcore_barrier()

        def iteration(n, p):
            q = 1 - p
            idx_wait(q)
            addoff(q)
            gather_start(q)

            @pl.when(n >= 2)
            def _():
                scatter_wait(p)

            gather_wait(p)

            qrp, krp, vrp, wrp = qr[p], kr[p], vr[p], wr[p]

            @plsc.parallel_loop(0, B, unroll=4)
            def _(i):
                h = (qrp[i, pl.ds(0, 16)] * krp[i, pl.ds(0, 16)]
                     + qrp[i, pl.ds(16, 16)] * krp[i, pl.ds(16, 16)])
                e = jnp.exp(jnp.broadcast_to(jnp.sum(h), (16,)))
                wrp[i, pl.ds(0, 16)] = vrp[i, pl.ds(0, 16)] * e
                wrp[i, pl.ds(16, 16)] = vrp[i, pl.ds(16, 16)] * e
                wrp[i, pl.ds(DH, 16)] = jnp.where(lanes < 1, e, zvec)

            goodv = jnp.broadcast_to(sid + n * NS < NCHUNK, (16,))
            for j in range(B // 16):
                sl = pl.ds(j * 16, 16)
                sc[p][sl] = jnp.where(goodv, idr[p][sl], junkv)
            pltpu.async_copy(wr[p], acc.at[sc[p]], ss[p], add=True)
            idx_start(n + 2, p)

        def pair(m, carry):
            iteration(2 * m, 0)
            iteration(2 * m + 1, 1)
            return carry

        lax.fori_loop(0, NLOCP // 2, pair, 0)

        scatter_wait(0)
        scatter_wait(1)
        gather_wait(0)
        idx_wait(1)
        plsc.subcore_barrier()
        pltpu.sync_copy(acc.at[pl.ds(row0, ROWS_PER)],
                        out_hbm.at[cid, pl.ds(row0, ROWS_PER)])

    pltpu.sync_copy(z_hbm, acc.at[pl.ds(row0, ROWS_PER)])
    run_phase(qdA, ksA, vsA, srcA, dstA, outA)
    pltpu.sync_copy(z_hbm, acc.at[pl.ds(row0, ROWS_PER)])
    run_phase(qdB, ksB, vsB, srcB, dstB, outB)


@functools.cache
def _edge_call_factory():
    return pl.kernel(
        _edge_body,
        out_type=[jax.ShapeDtypeStruct((NC, NP, AW), jnp.float32)] * 2,
        mesh=plsc.VectorSubcoreMesh(core_axis_name="c", subcore_axis_name="s",
                                    num_cores=NC, num_subcores=NS),
        compiler_params=pltpu.CompilerParams(needs_layout_passes=False,
                                             use_tc_tiling_on_sc=False),
        scratch_types=(
            [pltpu.VMEM((B,), jnp.int32)] * 10
            + [pltpu.VMEM((B, DH), jnp.float32)] * 6
            + [pltpu.VMEM((B, AW), jnp.float32)] * 2
            + [pltpu.VMEM_SHARED((NP, AW), jnp.float32)]
            + [pltpu.SemaphoreType.DMA] * 10
        ),
    )


def _blockdiag(a, scale):
    z = jnp.zeros((C, C), jnp.float32)
    z = z.at[0:DH, 0:DH].set(a[0] * scale[0])
    z = z.at[DH:C, DH:C].set(a[1] * scale[1])
    return z


def _fold(p, rel):
    s = rel["p"] / math.sqrt(DH)
    ablk = _blockdiag(rel["a"], s)
    mblk = _blockdiag(rel["m"], jnp.ones((H,), jnp.float32))
    wq, wk, wv = p["Wq"], p["Wk"] @ ablk, p["Wv"] @ mblk
    bq, bk, bv = p["bq"], p["bk"] @ ablk, p["bv"] @ mblk
    wcat = jnp.stack([jnp.concatenate([wq[:, 0:DH], wk[:, 0:DH], wv[:, 0:DH]], axis=1),
                      jnp.concatenate([wq[:, DH:C], wk[:, DH:C], wv[:, DH:C]], axis=1)])
    bcat = jnp.stack([jnp.concatenate([bq[0:DH], bk[0:DH], bv[0:DH]]),
                      jnp.concatenate([bq[DH:C], bk[DH:C], bv[DH:C]])])
    lnw = jnp.stack([p["b_in"], p["g_in"], p["b_ln_in"]])
    return wcat, bcat.reshape(H, 1, 3 * DH), lnw


def _misc(p):
    sig = jax.nn.sigmoid(p["skip"])
    return jnp.stack([p["bo"], jnp.full((C,), sig), p["g_out"], p["b_out"]])


def kernel(x_user, x_item, params, ei_user_rates_item, ei_item_rev_rates_user):
    pu, pi = params["user"], params["item"]
    ru, ri = params["rel"]["u2i"], params["rel"]["i2u"]

    wcat_u, bcat_u, lnw_u = _fold(pu, ru)
    wcat_i, bcat_i, lnw_i = _fold(pi, ri)

    xs_u, q_u, krel_u, vrel_u = _pre_call(x_user, pu["W_in"], lnw_u, wcat_u, bcat_u)
    xs_i, q_i, krel_i, vrel_i = _pre_call(x_item, pi["W_in"], lnw_i, wcat_i, bcat_i)

    z = jnp.zeros((ROWS_PER, AW), jnp.float32)
    src_ui = ei_user_rates_item[0].astype(jnp.int32)
    dst_ui = ei_user_rates_item[1].astype(jnp.int32)
    src_iu = ei_item_rev_rates_user[0].astype(jnp.int32)
    dst_iu = ei_item_rev_rates_user[1].astype(jnp.int32)

    edge = _edge_call_factory()
    acc_item, acc_user = edge(q_i, krel_u, vrel_u, src_ui, dst_ui,
                              q_u, krel_i, vrel_i, src_iu, dst_iu, z)

    y_user = _post_call(acc_user, xs_u, pu["Wo"], _misc(pu))
    y_item = _post_call(acc_item, xs_i, pi["Wo"], _misc(pi))
    return y_user, y_item


# speedup vs baseline: 1.0380x; 1.0380x over previous
import functools
import math

import jax
import jax.numpy as jnp
from jax import lax
from jax.experimental import pallas as pl
from jax.experimental.pallas import tpu as pltpu
from jax.experimental.pallas import tpu_sc as plsc

N = 25000
D = 128
C = 64
H = 2
DH = 32
E = 400000

NC = 2
NS = 16
NW = NC * NS
B = 128
NCHUNK = E // B
NP = 25088
ROWS_PER = NP // NS
AW = 48
NLOCP = 2 * ((NCHUNK + 2 * NS - 1) // (2 * NS))
JUNK = NP - 8

R = 1000
GRID = N // R


def _pre_body(x_ref, win_ref, lnw_ref, wcat_ref, bcat_ref,
              xs_ref, q_ref, k_ref, v_ref):
    h = jnp.dot(x_ref[...], win_ref[...], preferred_element_type=jnp.float32)
    h = h + lnw_ref[0:1, :]
    m = jnp.mean(h, axis=-1, keepdims=True)
    v = jnp.mean((h - m) * (h - m), axis=-1, keepdims=True)
    hn = (h - m) / jnp.sqrt(v + 1e-5) * lnw_ref[1:2, :] + lnw_ref[2:3, :]
    xs = jnp.maximum(hn, 0.0)
    qkv = jnp.dot(xs, wcat_ref[0], preferred_element_type=jnp.float32)
    qkv = qkv + bcat_ref[0]
    xs_ref[...] = xs
    q_ref[...] = qkv[:, 0:DH].astype(jnp.bfloat16)
    k_ref[...] = qkv[:, DH:2 * DH].astype(jnp.bfloat16)
    v_ref[...] = qkv[:, 2 * DH:3 * DH]


_pre_call = pl.pallas_call(
    _pre_body,
    grid=(H, GRID),
    in_specs=[
        pl.BlockSpec((R, D), lambda h, i: (i, 0)),
        pl.BlockSpec((D, C), lambda h, i: (0, 0)),
        pl.BlockSpec((3, C), lambda h, i: (0, 0)),
        pl.BlockSpec((1, C, 3 * DH), lambda h, i: (h, 0, 0)),
        pl.BlockSpec((1, 1, 3 * DH), lambda h, i: (h, 0, 0)),
    ],
    out_specs=[pl.BlockSpec((R, C), lambda h, i: (i, 0))]
    + [pl.BlockSpec((R, DH), lambda h, i: (h * GRID + i, 0))] * 3,
    out_shape=[jax.ShapeDtypeStruct((N, C), jnp.float32),
               jax.ShapeDtypeStruct((H * N, DH), jnp.bfloat16),
               jax.ShapeDtypeStruct((H * N, DH), jnp.bfloat16),
               jax.ShapeDtypeStruct((H * N, DH), jnp.float32)],
)


def _post_body(acc_ref, xs_ref, wo_ref, misc_ref, y_ref):
    m0 = acc_ref[0, :, 0:DH]
    m1 = acc_ref[1, :, 0:DH]
    d0 = acc_ref[0, :, DH:DH + 1]
    d1 = acc_ref[1, :, DH:DH + 1]
    msg = jnp.concatenate([m0 / (d0 + 1e-16), m1 / (d1 + 1e-16)], axis=1)
    o = jnp.dot(jax.nn.gelu(msg), wo_ref[...],
                preferred_element_type=jnp.float32) + misc_ref[0:1, :]
    xs = xs_ref[...]
    sig = misc_ref[1:2, :]
    out = sig * o + (1.0 - sig) * xs
    t = out + xs
    m = jnp.mean(t, axis=-1, keepdims=True)
    v = jnp.mean((t - m) * (t - m), axis=-1, keepdims=True)
    y_ref[...] = (t - m) / jnp.sqrt(v + 1e-5) * misc_ref[2:3, :] + misc_ref[3:4, :]


_post_call = pl.pallas_call(
    _post_body,
    grid=(GRID,),
    in_specs=[
        pl.BlockSpec((2, R, AW), lambda i: (0, i, 0)),
        pl.BlockSpec((R, C), lambda i: (i, 0)),
        pl.BlockSpec((C, C), lambda i: (0, 0)),
        pl.BlockSpec((4, C), lambda i: (0, 0)),
    ],
    out_specs=pl.BlockSpec((R, C), lambda i: (i, 0)),
    out_shape=jax.ShapeDtypeStruct((N, C), jnp.float32),
)


def _edge_body(qdA, ksA, vsA, srcA, dstA, qdB, ksB, vsB, srcB, dstB,
               z_hbm, outA, outB,
               isr0, isr1, idr0, idr1, ig0, ig1, is0, is1, sc0, sc1,
               qr0, qr1, kr0, kr1, vr0, vr1, wr0, wr1, acc,
               si0, si1, sq0, sq1, sk0, sk1, sv0, sv1, ss0, ss1):
    isr = (isr0, isr1)
    idr = (idr0, idr1)
    ig = (ig0, ig1)
    isx = (is0, is1)
    sc = (sc0, sc1)
    qr = (qr0, qr1)
    kr = (kr0, kr1)
    vr = (vr0, vr1)
    wr = (wr0, wr1)
    si = (si0, si1)
    sq = (sq0, sq1)
    sk = (sk0, sk1)
    sv = (sv0, sv1)
    ss = (ss0, ss1)

    cid = lax.axis_index("c")
    sid = lax.axis_index("s")
    row0 = sid * ROWS_PER

    offv = jnp.broadcast_to(cid * N, (16,))
    lanes = lax.iota(jnp.int32, 16)
    zvec = jnp.zeros((16,), jnp.float32)
    junkv = jnp.full((16,), JUNK, jnp.int32)

    def run_phase(qd_hbm, ks_hbm, vs_hbm, src_hbm, dst_hbm, out_hbm):
        def base_of(m):
            return jnp.minimum(sid + m * NS, NCHUNK - 1) * B

        def idx_start(m, b):
            ba = base_of(m)
            pltpu.async_copy(src_hbm.at[pl.ds(ba, B)], isr[b], si[b])
            pltpu.async_copy(dst_hbm.at[pl.ds(ba, B)], idr[b], si[b])

        def idx_wait(b):
            pltpu.make_async_copy(src_hbm.at[pl.ds(0, B)], isr[b], si[b]).wait()
            pltpu.make_async_copy(dst_hbm.at[pl.ds(0, B)], idr[b], si[b]).wait()

        def addoff(b):
            for j in range(B // 16):
                sl = pl.ds(j * 16, 16)
                ig[b][sl] = idr[b][sl] + offv
                isx[b][sl] = isr[b][sl] + offv

        def gather_start(b):
            pltpu.async_copy(qd_hbm.at[ig[b]], qr[b], sq[b])
            pltpu.async_copy(ks_hbm.at[isx[b]], kr[b], sk[b])
            pltpu.async_copy(vs_hbm.at[isx[b]], vr[b], sv[b])

        def gather_wait(b):
            pltpu.make_async_copy(qd_hbm.at[ig[b]], qr[b], sq[b]).wait()
            pltpu.make_async_copy(ks_hbm.at[isx[b]], kr[b], sk[b]).wait()
            pltpu.make_async_copy(vs_hbm.at[isx[b]], vr[b], sv[b]).wait()

        def scatter_wait(b):
            pltpu.make_async_copy(wr[b], acc.at[sc[b]], ss[b]).wait()

        idx_start(0, 0)
        idx_wait(0)
        addoff(0)
        gather_start(0)
        idx_start(1, 1)
        plsc.subcore_barrier()

        def iteration(n, p):
            q = 1 - p
            idx_wait(q)
            addoff(q)
            gather_start(q)

            @pl.when(n >= 2)
            def _():
                scatter_wait(p)

            gather_wait(p)

            qrp, krp, vrp, wrp = qr[p], kr[p], vr[p], wr[p]

            @plsc.parallel_loop(0, B, unroll=4)
            def _(i):
                qa, qb = plsc.unpack(qrp[i, pl.ds(0, DH)],
                                     format=plsc.PackFormat.INTERLEAVED)
                ka, kb = plsc.unpack(krp[i, pl.ds(0, DH)],
                                     format=plsc.PackFormat.INTERLEAVED)
                h = qa * ka + qb * kb
                e = jnp.exp(jnp.broadcast_to(jnp.sum(h), (16,)))
                wrp[i, pl.ds(0, 16)] = vrp[i, pl.ds(0, 16)] * e
                wrp[i, pl.ds(16, 16)] = vrp[i, pl.ds(16, 16)] * e
                wrp[i, pl.ds(DH, 16)] = jnp.where(lanes < 1, e, zvec)

            goodv = jnp.broadcast_to(sid + n * NS < NCHUNK, (16,))
            for j in range(B // 16):
                sl = pl.ds(j * 16, 16)
                sc[p][sl] = jnp.where(goodv, idr[p][sl], junkv)
            pltpu.async_copy(wr[p], acc.at[sc[p]], ss[p], add=True)
            idx_start(n + 2, p)

        def pair(m, carry):
            iteration(2 * m, 0)
            iteration(2 * m + 1, 1)
            return carry

        lax.fori_loop(0, NLOCP // 2, pair, 0)

        scatter_wait(0)
        scatter_wait(1)
        gather_wait(0)
        idx_wait(1)
        plsc.subcore_barrier()
        pltpu.sync_copy(acc.at[pl.ds(row0, ROWS_PER)],
                        out_hbm.at[cid, pl.ds(row0, ROWS_PER)])

    pltpu.sync_copy(z_hbm, acc.at[pl.ds(row0, ROWS_PER)])
    run_phase(qdA, ksA, vsA, srcA, dstA, outA)
    pltpu.sync_copy(z_hbm, acc.at[pl.ds(row0, ROWS_PER)])
    run_phase(qdB, ksB, vsB, srcB, dstB, outB)


@functools.cache
def _edge_call_factory():
    return pl.kernel(
        _edge_body,
        out_type=[jax.ShapeDtypeStruct((NC, NP, AW), jnp.float32)] * 2,
        mesh=plsc.VectorSubcoreMesh(core_axis_name="c", subcore_axis_name="s",
                                    num_cores=NC, num_subcores=NS),
        compiler_params=pltpu.CompilerParams(needs_layout_passes=False,
                                             use_tc_tiling_on_sc=False),
        scratch_types=(
            [pltpu.VMEM((B,), jnp.int32)] * 10
            + [pltpu.VMEM((B, DH), jnp.bfloat16)] * 4
            + [pltpu.VMEM((B, DH), jnp.float32)] * 2
            + [pltpu.VMEM((B, AW), jnp.float32)] * 2
            + [pltpu.VMEM_SHARED((NP, AW), jnp.float32)]
            + [pltpu.SemaphoreType.DMA] * 10
        ),
    )


def _blockdiag(a, scale):
    z = jnp.zeros((C, C), jnp.float32)
    z = z.at[0:DH, 0:DH].set(a[0] * scale[0])
    z = z.at[DH:C, DH:C].set(a[1] * scale[1])
    return z


def _fold(p, rel):
    s = rel["p"] / math.sqrt(DH)
    ablk = _blockdiag(rel["a"], s)
    mblk = _blockdiag(rel["m"], jnp.ones((H,), jnp.float32))
    wq, wk, wv = p["Wq"], p["Wk"] @ ablk, p["Wv"] @ mblk
    bq, bk, bv = p["bq"], p["bk"] @ ablk, p["bv"] @ mblk
    wcat = jnp.stack([jnp.concatenate([wq[:, 0:DH], wk[:, 0:DH], wv[:, 0:DH]], axis=1),
                      jnp.concatenate([wq[:, DH:C], wk[:, DH:C], wv[:, DH:C]], axis=1)])
    bcat = jnp.stack([jnp.concatenate([bq[0:DH], bk[0:DH], bv[0:DH]]),
                      jnp.concatenate([bq[DH:C], bk[DH:C], bv[DH:C]])])
    lnw = jnp.stack([p["b_in"], p["g_in"], p["b_ln_in"]])
    return wcat, bcat.reshape(H, 1, 3 * DH), lnw


def _misc(p):
    sig = jax.nn.sigmoid(p["skip"])
    return jnp.stack([p["bo"], jnp.full((C,), sig), p["g_out"], p["b_out"]])


def kernel(x_user, x_item, params, ei_user_rates_item, ei_item_rev_rates_user):
    pu, pi = params["user"], params["item"]
    ru, ri = params["rel"]["u2i"], params["rel"]["i2u"]

    wcat_u, bcat_u, lnw_u = _fold(pu, ru)
    wcat_i, bcat_i, lnw_i = _fold(pi, ri)

    xs_u, q_u, krel_u, vrel_u = _pre_call(x_user, pu["W_in"], lnw_u, wcat_u, bcat_u)
    xs_i, q_i, krel_i, vrel_i = _pre_call(x_item, pi["W_in"], lnw_i, wcat_i, bcat_i)

    z = jnp.zeros((ROWS_PER, AW), jnp.float32)
    src_ui = ei_user_rates_item[0].astype(jnp.int32)
    dst_ui = ei_user_rates_item[1].astype(jnp.int32)
    src_iu = ei_item_rev_rates_user[0].astype(jnp.int32)
    dst_iu = ei_item_rev_rates_user[1].astype(jnp.int32)

    edge = _edge_call_factory()
    acc_item, acc_user = edge(q_i, krel_u, vrel_u, src_ui, dst_ui,
                              q_u, krel_i, vrel_i, src_iu, dst_iu, z)

    y_user = _post_call(acc_user, xs_u, pu["Wo"], _misc(pu))
    y_item = _post_call(acc_item, xs_i, pi["Wo"], _misc(pi))
    return y_user, y_item


# speedup vs baseline: 1.1443x; 1.1024x over previous
import functools
import math

import jax
import jax.numpy as jnp
from jax import lax
from jax.experimental import pallas as pl
from jax.experimental.pallas import tpu as pltpu
from jax.experimental.pallas import tpu_sc as plsc

N = 25000
D = 128
C = 64
H = 2
DH = 32
E = 400000

NC = 2
NS = 16
NW = NC * NS
B = 128
NCHUNK = E // B
NP = 25088
ROWS_PER = NP // NS
AW = 48
NLOCP = 2 * ((NCHUNK + 2 * NS - 1) // (2 * NS))
JUNK = NP - 8

R = 1000
GRID = N // R


def _pre_body(x_ref, win_ref, lnw_ref, wcat_ref, bcat_ref,
              xs_ref, q_ref, k_ref, v_ref):
    h = jnp.dot(x_ref[...], win_ref[...], preferred_element_type=jnp.float32)
    h = h + lnw_ref[0:1, :]
    m = jnp.mean(h, axis=-1, keepdims=True)
    v = jnp.mean((h - m) * (h - m), axis=-1, keepdims=True)
    hn = (h - m) / jnp.sqrt(v + 1e-5) * lnw_ref[1:2, :] + lnw_ref[2:3, :]
    xs = jnp.maximum(hn, 0.0)
    qkv = jnp.dot(xs, wcat_ref[0], preferred_element_type=jnp.float32)
    qkv = qkv + bcat_ref[0]
    xs_ref[...] = xs
    q_ref[...] = qkv[:, 0:DH].astype(jnp.bfloat16)
    k_ref[...] = qkv[:, DH:2 * DH].astype(jnp.bfloat16)
    v_ref[...] = qkv[:, 2 * DH:3 * DH]


_pre_call = pl.pallas_call(
    _pre_body,
    grid=(H, GRID),
    in_specs=[
        pl.BlockSpec((R, D), lambda h, i: (i, 0)),
        pl.BlockSpec((D, C), lambda h, i: (0, 0)),
        pl.BlockSpec((3, C), lambda h, i: (0, 0)),
        pl.BlockSpec((1, C, 3 * DH), lambda h, i: (h, 0, 0)),
        pl.BlockSpec((1, 1, 3 * DH), lambda h, i: (h, 0, 0)),
    ],
    out_specs=[pl.BlockSpec((R, C), lambda h, i: (i, 0))]
    + [pl.BlockSpec((R, DH), lambda h, i: (h * GRID + i, 0))] * 3,
    out_shape=[jax.ShapeDtypeStruct((N, C), jnp.float32),
               jax.ShapeDtypeStruct((H * N, DH), jnp.bfloat16),
               jax.ShapeDtypeStruct((H * N, DH), jnp.bfloat16),
               jax.ShapeDtypeStruct((H * N, DH), jnp.float32)],
)


def _post_body(acc_ref, xs_ref, wo_ref, misc_ref, y_ref):
    m0 = acc_ref[0, :, 0:DH]
    m1 = acc_ref[1, :, 0:DH]
    d0 = acc_ref[0, :, DH:DH + 1]
    d1 = acc_ref[1, :, DH:DH + 1]
    msg = jnp.concatenate([m0 / (d0 + 1e-16), m1 / (d1 + 1e-16)], axis=1)
    o = jnp.dot(jax.nn.gelu(msg), wo_ref[...],
                preferred_element_type=jnp.float32) + misc_ref[0:1, :]
    xs = xs_ref[...]
    sig = misc_ref[1:2, :]
    out = sig * o + (1.0 - sig) * xs
    t = out + xs
    m = jnp.mean(t, axis=-1, keepdims=True)
    v = jnp.mean((t - m) * (t - m), axis=-1, keepdims=True)
    y_ref[...] = (t - m) / jnp.sqrt(v + 1e-5) * misc_ref[2:3, :] + misc_ref[3:4, :]


_post_call = pl.pallas_call(
    _post_body,
    grid=(GRID,),
    in_specs=[
        pl.BlockSpec((2, R, AW), lambda i: (0, i, 0)),
        pl.BlockSpec((R, C), lambda i: (i, 0)),
        pl.BlockSpec((C, C), lambda i: (0, 0)),
        pl.BlockSpec((4, C), lambda i: (0, 0)),
    ],
    out_specs=pl.BlockSpec((R, C), lambda i: (i, 0)),
    out_shape=jax.ShapeDtypeStruct((N, C), jnp.float32),
)


def _edge_body(qdA, ksA, vsA, srcA, dstA, qdB, ksB, vsB, srcB, dstB,
               z_hbm, outA, outB,
               isr0, isr1, idr0, idr1, ig0, ig1, is0, is1, sc0, sc1,
               qr0, qr1, kr0, kr1, vr0, vr1, wr0, wr1, acc,
               si0, si1, sq0, sq1, sk0, sk1, sv0, sv1, ss0, ss1):
    isr = (isr0, isr1)
    idr = (idr0, idr1)
    ig = (ig0, ig1)
    isx = (is0, is1)
    sc = (sc0, sc1)
    qr = (qr0, qr1)
    kr = (kr0, kr1)
    vr = (vr0, vr1)
    wr = (wr0, wr1)
    si = (si0, si1)
    sq = (sq0, sq1)
    sk = (sk0, sk1)
    sv = (sv0, sv1)
    ss = (ss0, ss1)

    cid = lax.axis_index("c")
    sid = lax.axis_index("s")
    row0 = sid * ROWS_PER

    offv = jnp.broadcast_to(cid * N, (16,))
    lanes = lax.iota(jnp.int32, 16)
    zvec = jnp.zeros((16,), jnp.float32)
    junkv = jnp.full((16,), JUNK, jnp.int32)

    def run_phase(qd_hbm, ks_hbm, vs_hbm, src_hbm, dst_hbm, out_hbm):
        def base_of(m):
            return jnp.minimum(sid + m * NS, NCHUNK - 1) * B

        def idx_start(m, b):
            ba = base_of(m)
            pltpu.async_copy(src_hbm.at[pl.ds(ba, B)], isr[b], si[b])
            pltpu.async_copy(dst_hbm.at[pl.ds(ba, B)], idr[b], si[b])

        def idx_wait(b):
            pltpu.make_async_copy(src_hbm.at[pl.ds(0, B)], isr[b], si[b]).wait()
            pltpu.make_async_copy(dst_hbm.at[pl.ds(0, B)], idr[b], si[b]).wait()

        def addoff(b):
            for j in range(B // 16):
                sl = pl.ds(j * 16, 16)
                ig[b][sl] = idr[b][sl] + offv
                isx[b][sl] = isr[b][sl] + offv

        def gather_start(b):
            pltpu.async_copy(qd_hbm.at[ig[b]], qr[b], sq[b])
            pltpu.async_copy(ks_hbm.at[isx[b]], kr[b], sk[b])
            pltpu.async_copy(vs_hbm.at[isx[b]], vr[b], sv[b])

        def gather_wait(b):
            pltpu.make_async_copy(qd_hbm.at[ig[b]], qr[b], sq[b]).wait()
            pltpu.make_async_copy(ks_hbm.at[isx[b]], kr[b], sk[b]).wait()
            pltpu.make_async_copy(vs_hbm.at[isx[b]], vr[b], sv[b]).wait()

        def scatter_wait(b):
            pltpu.make_async_copy(wr[b], acc.at[sc[b]], ss[b]).wait()

        idx_start(0, 0)
        idx_wait(0)
        addoff(0)
        gather_start(0)
        idx_start(1, 1)
        plsc.subcore_barrier()

        def iteration(n, p):
            q = 1 - p
            idx_wait(q)
            addoff(q)
            gather_start(q)

            @pl.when(n >= 2)
            def _():
                scatter_wait(p)

            goodv = jnp.broadcast_to(sid + n * NS < NCHUNK, (16,))
            for j in range(B // 16):
                sl = pl.ds(j * 16, 16)
                sc[p][sl] = jnp.where(goodv, idr[p][sl], junkv)
            idx_start(n + 2, p)

            gather_wait(p)

            qrp, krp, vrp, wrp = qr[p], kr[p], vr[p], wr[p]

            @plsc.parallel_loop(0, B, unroll=4)
            def _(i):
                qa, qb = plsc.unpack(qrp[i, pl.ds(0, DH)],
                                     format=plsc.PackFormat.INTERLEAVED)
                ka, kb = plsc.unpack(krp[i, pl.ds(0, DH)],
                                     format=plsc.PackFormat.INTERLEAVED)
                h = qa * ka + qb * kb
                e = jnp.exp(jnp.broadcast_to(jnp.sum(h), (16,)))
                wrp[i, pl.ds(0, 16)] = vrp[i, pl.ds(0, 16)] * e
                wrp[i, pl.ds(16, 16)] = vrp[i, pl.ds(16, 16)] * e
                wrp[i, pl.ds(DH, 16)] = jnp.where(lanes < 1, e, zvec)

            pltpu.async_copy(wr[p], acc.at[sc[p]], ss[p], add=True)

        def pair(m, carry):
            iteration(2 * m, 0)
            iteration(2 * m + 1, 1)
            return carry

        lax.fori_loop(0, NLOCP // 2, pair, 0)

        scatter_wait(0)
        scatter_wait(1)
        gather_wait(0)
        idx_wait(1)
        plsc.subcore_barrier()
        pltpu.sync_copy(acc.at[pl.ds(row0, ROWS_PER)],
                        out_hbm.at[cid, pl.ds(row0, ROWS_PER)])

    pltpu.sync_copy(z_hbm, acc.at[pl.ds(row0, ROWS_PER)])
    run_phase(qdA, ksA, vsA, srcA, dstA, outA)
    pltpu.sync_copy(z_hbm, acc.at[pl.ds(row0, ROWS_PER)])
    run_phase(qdB, ksB, vsB, srcB, dstB, outB)


@functools.cache
def _edge_call_factory():
    return pl.kernel(
        _edge_body,
        out_type=[jax.ShapeDtypeStruct((NC, NP, AW), jnp.float32)] * 2,
        mesh=plsc.VectorSubcoreMesh(core_axis_name="c", subcore_axis_name="s",
                                    num_cores=NC, num_subcores=NS),
        compiler_params=pltpu.CompilerParams(needs_layout_passes=False,
                                             use_tc_tiling_on_sc=False),
        scratch_types=(
            [pltpu.VMEM((B,), jnp.int32)] * 10
            + [pltpu.VMEM((B, DH), jnp.bfloat16)] * 4
            + [pltpu.VMEM((B, DH), jnp.float32)] * 2
            + [pltpu.VMEM((B, AW), jnp.float32)] * 2
            + [pltpu.VMEM_SHARED((NP, AW), jnp.float32)]
            + [pltpu.SemaphoreType.DMA] * 10
        ),
    )


def _blockdiag(a, scale):
    z = jnp.zeros((C, C), jnp.float32)
    z = z.at[0:DH, 0:DH].set(a[0] * scale[0])
    z = z.at[DH:C, DH:C].set(a[1] * scale[1])
    return z


def _fold(p, rel):
    s = rel["p"] / math.sqrt(DH)
    ablk = _blockdiag(rel["a"], s)
    mblk = _blockdiag(rel["m"], jnp.ones((H,), jnp.float32))
    wq, wk, wv = p["Wq"], p["Wk"] @ ablk, p["Wv"] @ mblk
    bq, bk, bv = p["bq"], p["bk"] @ ablk, p["bv"] @ mblk
    wcat = jnp.stack([jnp.concatenate([wq[:, 0:DH], wk[:, 0:DH], wv[:, 0:DH]], axis=1),
                      jnp.concatenate([wq[:, DH:C], wk[:, DH:C], wv[:, DH:C]], axis=1)])
    bcat = jnp.stack([jnp.concatenate([bq[0:DH], bk[0:DH], bv[0:DH]]),
                      jnp.concatenate([bq[DH:C], bk[DH:C], bv[DH:C]])])
    lnw = jnp.stack([p["b_in"], p["g_in"], p["b_ln_in"]])
    return wcat, bcat.reshape(H, 1, 3 * DH), lnw


def _misc(p):
    sig = jax.nn.sigmoid(p["skip"])
    return jnp.stack([p["bo"], jnp.full((C,), sig), p["g_out"], p["b_out"]])


def kernel(x_user, x_item, params, ei_user_rates_item, ei_item_rev_rates_user):
    pu, pi = params["user"], params["item"]
    ru, ri = params["rel"]["u2i"], params["rel"]["i2u"]

    wcat_u, bcat_u, lnw_u = _fold(pu, ru)
    wcat_i, bcat_i, lnw_i = _fold(pi, ri)

    xs_u, q_u, krel_u, vrel_u = _pre_call(x_user, pu["W_in"], lnw_u, wcat_u, bcat_u)
    xs_i, q_i, krel_i, vrel_i = _pre_call(x_item, pi["W_in"], lnw_i, wcat_i, bcat_i)

    z = jnp.zeros((ROWS_PER, AW), jnp.float32)
    src_ui = ei_user_rates_item[0].astype(jnp.int32)
    dst_ui = ei_user_rates_item[1].astype(jnp.int32)
    src_iu = ei_item_rev_rates_user[0].astype(jnp.int32)
    dst_iu = ei_item_rev_rates_user[1].astype(jnp.int32)

    edge = _edge_call_factory()
    acc_item, acc_user = edge(q_i, krel_u, vrel_u, src_ui, dst_ui,
                              q_u, krel_i, vrel_i, src_iu, dst_iu, z)

    y_user = _post_call(acc_user, xs_u, pu["Wo"], _misc(pu))
    y_item = _post_call(acc_item, xs_i, pi["Wo"], _misc(pi))
    return y_user, y_item


# speedup vs baseline: 1.2380x; 1.0819x over previous
import functools
import math

import jax
import jax.numpy as jnp
from jax import lax
from jax.experimental import pallas as pl
from jax.experimental.pallas import tpu as pltpu
from jax.experimental.pallas import tpu_sc as plsc

N = 25000
D = 128
C = 64
H = 2
DH = 32
E = 400000

NC = 2
NS = 16
NW = NC * NS
B = 256
HB = 128
NCHUNK = (E + B - 1) // B
NP = 25088
ROWS_PER = NP // NS
AW = 48
NLOCP = 2 * ((NCHUNK + 2 * NS - 1) // (2 * NS))
JUNK = NP - 8

R = 1000
GRID = N // R


def _pre_body(x_ref, win_ref, lnw_ref, wcat_ref, bcat_ref,
              xs_ref, q_ref, k_ref, v_ref):
    h = jnp.dot(x_ref[...], win_ref[...], preferred_element_type=jnp.float32)
    h = h + lnw_ref[0:1, :]
    m = jnp.mean(h, axis=-1, keepdims=True)
    v = jnp.mean((h - m) * (h - m), axis=-1, keepdims=True)
    hn = (h - m) / jnp.sqrt(v + 1e-5) * lnw_ref[1:2, :] + lnw_ref[2:3, :]
    xs = jnp.maximum(hn, 0.0)
    qkv = jnp.dot(xs, wcat_ref[0], preferred_element_type=jnp.float32)
    qkv = qkv + bcat_ref[0]
    xs_ref[...] = xs
    q_ref[...] = qkv[:, 0:DH].astype(jnp.bfloat16)
    k_ref[...] = qkv[:, DH:2 * DH].astype(jnp.bfloat16)
    v_ref[...] = qkv[:, 2 * DH:3 * DH].astype(jnp.bfloat16)


_pre_call = pl.pallas_call(
    _pre_body,
    grid=(H, GRID),
    in_specs=[
        pl.BlockSpec((R, D), lambda h, i: (i, 0)),
        pl.BlockSpec((D, C), lambda h, i: (0, 0)),
        pl.BlockSpec((3, C), lambda h, i: (0, 0)),
        pl.BlockSpec((1, C, 3 * DH), lambda h, i: (h, 0, 0)),
        pl.BlockSpec((1, 1, 3 * DH), lambda h, i: (h, 0, 0)),
    ],
    out_specs=[pl.BlockSpec((R, C), lambda h, i: (i, 0))]
    + [pl.BlockSpec((R, DH), lambda h, i: (h * GRID + i, 0))] * 3,
    out_shape=[jax.ShapeDtypeStruct((N, C), jnp.float32)]
    + [jax.ShapeDtypeStruct((H * N, DH), jnp.bfloat16)] * 3,
)


def _post_body(acc_ref, xs_ref, wo_ref, misc_ref, y_ref):
    m0 = acc_ref[0, :, 0:DH]
    m1 = acc_ref[1, :, 0:DH]
    d0 = acc_ref[0, :, DH:DH + 1]
    d1 = acc_ref[1, :, DH:DH + 1]
    msg = jnp.concatenate([m0 / (d0 + 1e-16), m1 / (d1 + 1e-16)], axis=1)
    o = jnp.dot(jax.nn.gelu(msg), wo_ref[...],
                preferred_element_type=jnp.float32) + misc_ref[0:1, :]
    xs = xs_ref[...]
    sig = misc_ref[1:2, :]
    out = sig * o + (1.0 - sig) * xs
    t = out + xs
    m = jnp.mean(t, axis=-1, keepdims=True)
    v = jnp.mean((t - m) * (t - m), axis=-1, keepdims=True)
    y_ref[...] = (t - m) / jnp.sqrt(v + 1e-5) * misc_ref[2:3, :] + misc_ref[3:4, :]


_post_call = pl.pallas_call(
    _post_body,
    grid=(GRID,),
    in_specs=[
        pl.BlockSpec((2, R, AW), lambda i: (0, i, 0)),
        pl.BlockSpec((R, C), lambda i: (i, 0)),
        pl.BlockSpec((C, C), lambda i: (0, 0)),
        pl.BlockSpec((4, C), lambda i: (0, 0)),
    ],
    out_specs=pl.BlockSpec((R, C), lambda i: (i, 0)),
    out_shape=jax.ShapeDtypeStruct((N, C), jnp.float32),
)


def _edge_body(qdA, ksA, vsA, srcA, dstA, qdB, ksB, vsB, srcB, dstB,
               z_hbm, outA, outB,
               isr0, isr1, idr0, idr1, iga0, iga1, igb0, igb1,
               isa0, isa1, isb0, isb1, sca0, sca1, scb0, scb1,
               qr0, qr1, kr0, kr1, vr0, vr1, wr0, wr1, acc,
               si0, si1, sq0, sq1, sk0, sk1, sv0, sv1, ss0, ss1):
    isr = (isr0, isr1)
    idr = (idr0, idr1)
    iga = (iga0, iga1)
    igb = (igb0, igb1)
    isa = (isa0, isa1)
    isb = (isb0, isb1)
    sca = (sca0, sca1)
    scb = (scb0, scb1)
    qr = (qr0, qr1)
    kr = (kr0, kr1)
    vr = (vr0, vr1)
    wr = (wr0, wr1)
    si = (si0, si1)
    sq = (sq0, sq1)
    sk = (sk0, sk1)
    sv = (sv0, sv1)
    ss = (ss0, ss1)

    cid = lax.axis_index("c")
    sid = lax.axis_index("s")
    row0 = sid * ROWS_PER

    offv = jnp.broadcast_to(cid * N, (16,))
    lanes = lax.iota(jnp.int32, 16)
    zvec = jnp.zeros((16,), jnp.float32)
    junkv = jnp.full((16,), JUNK, jnp.int32)

    def run_phase(qd_hbm, ks_hbm, vs_hbm, src_hbm, dst_hbm, out_hbm):
        def idx_start(m, b):
            ba = jnp.minimum(sid + m * NS, NCHUNK - 1) * B
            bb = jnp.minimum(ba + HB, E - HB)
            pltpu.async_copy(src_hbm.at[pl.ds(ba, HB)],
                             isr[b].at[pl.ds(0, HB)], si[b])
            pltpu.async_copy(src_hbm.at[pl.ds(bb, HB)],
                             isr[b].at[pl.ds(HB, HB)], si[b])
            pltpu.async_copy(dst_hbm.at[pl.ds(ba, HB)],
                             idr[b].at[pl.ds(0, HB)], si[b])
            pltpu.async_copy(dst_hbm.at[pl.ds(bb, HB)],
                             idr[b].at[pl.ds(HB, HB)], si[b])

        def idx_wait(b):
            for rf in (isr[b], idr[b]):
                for o in (0, HB):
                    pltpu.make_async_copy(src_hbm.at[pl.ds(0, HB)],
                                          rf.at[pl.ds(o, HB)], si[b]).wait()

        def addoff(b):
            for j in range(B // 16):
                sl = pl.ds(j * 16, 16)
                sl2 = pl.ds((j % 8) * 16, 16)
                tg = iga[b] if j < 8 else igb[b]
                ts = isa[b] if j < 8 else isb[b]
                tg[sl2] = idr[b][sl] + offv
                ts[sl2] = isr[b][sl] + offv

        def gather_start(b):
            pltpu.async_copy(qd_hbm.at[iga[b]], qr[b].at[pl.ds(0, HB)], sq[b])
            pltpu.async_copy(qd_hbm.at[igb[b]], qr[b].at[pl.ds(HB, HB)], sq[b])
            pltpu.async_copy(ks_hbm.at[isa[b]], kr[b].at[pl.ds(0, HB)], sk[b])
            pltpu.async_copy(ks_hbm.at[isb[b]], kr[b].at[pl.ds(HB, HB)], sk[b])
            pltpu.async_copy(vs_hbm.at[isa[b]], vr[b].at[pl.ds(0, HB)], sv[b])
            pltpu.async_copy(vs_hbm.at[isb[b]], vr[b].at[pl.ds(HB, HB)], sv[b])

        def gather_wait(b):
            pltpu.make_async_copy(qd_hbm.at[iga[b]], qr[b].at[pl.ds(0, HB)], sq[b]).wait()
            pltpu.make_async_copy(qd_hbm.at[igb[b]], qr[b].at[pl.ds(HB, HB)], sq[b]).wait()
            pltpu.make_async_copy(ks_hbm.at[isa[b]], kr[b].at[pl.ds(0, HB)], sk[b]).wait()
            pltpu.make_async_copy(ks_hbm.at[isb[b]], kr[b].at[pl.ds(HB, HB)], sk[b]).wait()
            pltpu.make_async_copy(vs_hbm.at[isa[b]], vr[b].at[pl.ds(0, HB)], sv[b]).wait()
            pltpu.make_async_copy(vs_hbm.at[isb[b]], vr[b].at[pl.ds(HB, HB)], sv[b]).wait()

        def scatter_wait(b):
            pltpu.make_async_copy(wr[b].at[pl.ds(0, HB)],
                                  acc.at[sca[b]], ss[b]).wait()
            pltpu.make_async_copy(wr[b].at[pl.ds(HB, HB)],
                                  acc.at[scb[b]], ss[b]).wait()

        idx_start(0, 0)
        idx_wait(0)
        addoff(0)
        gather_start(0)
        idx_start(1, 1)
        plsc.subcore_barrier()

        def iteration(n, p):
            q = 1 - p
            idx_wait(q)
            addoff(q)
            gather_start(q)

            @pl.when(n >= 2)
            def _():
                scatter_wait(p)

            e0 = (sid + n * NS) * B
            for j in range(B // 16):
                sl = pl.ds(j * 16, 16)
                sl2 = pl.ds((j % 8) * 16, 16)
                tgt = sca[p] if j < 8 else scb[p]
                egv = lanes + (e0 + j * 16)
                tgt[sl2] = jnp.where(egv < E, idr[p][sl], junkv)
            idx_start(n + 2, p)

            gather_wait(p)

            qrp, krp, vrp, wrp = qr[p], kr[p], vr[p], wr[p]

            @plsc.parallel_loop(0, B, unroll=4)
            def _(i):
                qa, qb = plsc.unpack(qrp[i, pl.ds(0, DH)],
                                     format=plsc.PackFormat.INTERLEAVED)
                ka, kb = plsc.unpack(krp[i, pl.ds(0, DH)],
                                     format=plsc.PackFormat.INTERLEAVED)
                h = qa * ka + qb * kb
                e = jnp.exp(jnp.broadcast_to(jnp.sum(h), (16,)))
                va, vb = plsc.unpack(vrp[i, pl.ds(0, DH)],
                                     format=plsc.PackFormat.INTERLEAVED)
                wrp[i, pl.ds(0, 16)] = va * e
                wrp[i, pl.ds(16, 16)] = vb * e
                wrp[i, pl.ds(DH, 16)] = jnp.where(lanes < 1, e, zvec)

            pltpu.async_copy(wr[p].at[pl.ds(0, HB)], acc.at[sca[p]],
                             ss[p], add=True)
            pltpu.async_copy(wr[p].at[pl.ds(HB, HB)], acc.at[scb[p]],
                             ss[p], add=True)

        def pair(m, carry):
            iteration(2 * m, 0)
            iteration(2 * m + 1, 1)
            return carry

        lax.fori_loop(0, NLOCP // 2, pair, 0)

        scatter_wait(0)
        scatter_wait(1)
        gather_wait(0)
        idx_wait(1)
        plsc.subcore_barrier()
        pltpu.sync_copy(acc.at[pl.ds(row0, ROWS_PER)],
                        out_hbm.at[cid, pl.ds(row0, ROWS_PER)])

    pltpu.sync_copy(z_hbm, acc.at[pl.ds(row0, ROWS_PER)])
    run_phase(qdA, ksA, vsA, srcA, dstA, outA)
    pltpu.sync_copy(z_hbm, acc.at[pl.ds(row0, ROWS_PER)])
    run_phase(qdB, ksB, vsB, srcB, dstB, outB)


@functools.cache
def _edge_call_factory():
    return pl.kernel(
        _edge_body,
        out_type=[jax.ShapeDtypeStruct((NC, NP, AW), jnp.float32)] * 2,
        mesh=plsc.VectorSubcoreMesh(core_axis_name="c", subcore_axis_name="s",
                                    num_cores=NC, num_subcores=NS),
        compiler_params=pltpu.CompilerParams(needs_layout_passes=False,
                                             use_tc_tiling_on_sc=False),
        scratch_types=(
            [pltpu.VMEM((B,), jnp.int32)] * 4
            + [pltpu.VMEM((HB,), jnp.int32)] * 12
            + [pltpu.VMEM((B, DH), jnp.bfloat16)] * 6
            + [pltpu.VMEM((B, AW), jnp.float32)] * 2
            + [pltpu.VMEM_SHARED((NP, AW), jnp.float32)]
            + [pltpu.SemaphoreType.DMA] * 10
        ),
    )


def _blockdiag(a, scale):
    z = jnp.zeros((C, C), jnp.float32)
    z = z.at[0:DH, 0:DH].set(a[0] * scale[0])
    z = z.at[DH:C, DH:C].set(a[1] * scale[1])
    return z


def _fold(p, rel):
    s = rel["p"] / math.sqrt(DH)
    ablk = _blockdiag(rel["a"], s)
    mblk = _blockdiag(rel["m"], jnp.ones((H,), jnp.float32))
    wq, wk, wv = p["Wq"], p["Wk"] @ ablk, p["Wv"] @ mblk
    bq, bk, bv = p["bq"], p["bk"] @ ablk, p["bv"] @ mblk

    def ileave_w(w):
        return jnp.stack([w[:, 0:DH // 2], w[:, DH // 2:DH]], axis=-1).reshape(C, DH)

    def ileave_b(b):
        return jnp.stack([b[0:DH // 2], b[DH // 2:DH]], axis=-1).reshape(DH)

    wv = jnp.concatenate([ileave_w(wv[:, 0:DH]), ileave_w(wv[:, DH:C])], axis=1)
    bv = jnp.concatenate([ileave_b(bv[0:DH]), ileave_b(bv[DH:C])])
    wcat = jnp.stack([jnp.concatenate([wq[:, 0:DH], wk[:, 0:DH], wv[:, 0:DH]], axis=1),
                      jnp.concatenate([wq[:, DH:C], wk[:, DH:C], wv[:, DH:C]], axis=1)])
    bcat = jnp.stack([jnp.concatenate([bq[0:DH], bk[0:DH], bv[0:DH]]),
                      jnp.concatenate([bq[DH:C], bk[DH:C], bv[DH:C]])])
    lnw = jnp.stack([p["b_in"], p["g_in"], p["b_ln_in"]])
    return wcat, bcat.reshape(H, 1, 3 * DH), lnw


def _misc(p):
    sig = jax.nn.sigmoid(p["skip"])
    return jnp.stack([p["bo"], jnp.full((C,), sig), p["g_out"], p["b_out"]])


def kernel(x_user, x_item, params, ei_user_rates_item, ei_item_rev_rates_user):
    pu, pi = params["user"], params["item"]
    ru, ri = params["rel"]["u2i"], params["rel"]["i2u"]

    wcat_u, bcat_u, lnw_u = _fold(pu, ru)
    wcat_i, bcat_i, lnw_i = _fold(pi, ri)

    xs_u, q_u, krel_u, vrel_u = _pre_call(x_user, pu["W_in"], lnw_u, wcat_u, bcat_u)
    xs_i, q_i, krel_i, vrel_i = _pre_call(x_item, pi["W_in"], lnw_i, wcat_i, bcat_i)

    z = jnp.zeros((ROWS_PER, AW), jnp.float32)
    src_ui = ei_user_rates_item[0].astype(jnp.int32)
    dst_ui = ei_user_rates_item[1].astype(jnp.int32)
    src_iu = ei_item_rev_rates_user[0].astype(jnp.int32)
    dst_iu = ei_item_rev_rates_user[1].astype(jnp.int32)

    edge = _edge_call_factory()
    acc_item, acc_user = edge(q_i, krel_u, vrel_u, src_ui, dst_ui,
                              q_u, krel_i, vrel_i, src_iu, dst_iu, z)

    y_user = _post_call(acc_user, xs_u, pu["Wo"], _misc(pu))
    y_item = _post_call(acc_item, xs_i, pi["Wo"], _misc(pi))
    return y_user, y_item


# speedup vs baseline: 1.2890x; 1.0411x over previous
import functools
import math

import jax
import jax.numpy as jnp
import numpy as np
from jax import lax
from jax.experimental import pallas as pl
from jax.experimental.pallas import tpu as pltpu
from jax.experimental.pallas import tpu_sc as plsc

N = 25000
D = 128
C = 64
H = 2
DH = 32
E = 400000

NC = 2
NS = 16
B = 256
HB = 128
NCHUNK = (E + B - 1) // B
NP = 25088
ROWS_PER = NP // NS
AW = 40
NLOCP = 2 * ((NCHUNK + 2 * NS - 1) // (2 * NS))
JUNK = NP - 8

R = 1000
GRID = N // R

_VP = np.array(list(range(0, 32, 2)) + list(range(1, 16, 2))
               + list(range(31, 16, -2)))
_VQ = np.empty(DH, np.int64)
_VQ[_VP] = np.arange(DH)


def _pre_body(x_ref, win_ref, lnw_ref, wcat_ref, bcat_ref,
              xs_ref, q_ref, kv_ref):
    h = jnp.dot(x_ref[...], win_ref[...], preferred_element_type=jnp.float32)
    h = h + lnw_ref[0:1, :]
    m = jnp.mean(h, axis=-1, keepdims=True)
    v = jnp.mean((h - m) * (h - m), axis=-1, keepdims=True)
    hn = (h - m) / jnp.sqrt(v + 1e-5) * lnw_ref[1:2, :] + lnw_ref[2:3, :]
    xs = jnp.maximum(hn, 0.0)
    qkv = jnp.dot(xs, wcat_ref[0], preferred_element_type=jnp.float32)
    qkv = qkv + bcat_ref[0]
    xs_ref[...] = xs
    q_ref[...] = qkv[:, 0:DH].astype(jnp.bfloat16)
    kv_ref[...] = qkv[:, DH:3 * DH].astype(jnp.bfloat16)


_pre_call = pl.pallas_call(
    _pre_body,
    grid=(H, GRID),
    in_specs=[
        pl.BlockSpec((R, D), lambda h, i: (i, 0)),
        pl.BlockSpec((D, C), lambda h, i: (0, 0)),
        pl.BlockSpec((3, C), lambda h, i: (0, 0)),
        pl.BlockSpec((1, C, 3 * DH), lambda h, i: (h, 0, 0)),
        pl.BlockSpec((1, 1, 3 * DH), lambda h, i: (h, 0, 0)),
    ],
    out_specs=[pl.BlockSpec((R, C), lambda h, i: (i, 0)),
               pl.BlockSpec((R, DH), lambda h, i: (h * GRID + i, 0)),
               pl.BlockSpec((R, 2 * DH), lambda h, i: (h * GRID + i, 0))],
    out_shape=[jax.ShapeDtypeStruct((N, C), jnp.float32),
               jax.ShapeDtypeStruct((H * N, DH), jnp.bfloat16),
               jax.ShapeDtypeStruct((H * N, 2 * DH), jnp.bfloat16)],
)


def _post_body(acc_ref, xs_ref, wo_ref, misc_ref, y_ref):
    m0 = acc_ref[0, :, 0:DH]
    m1 = acc_ref[1, :, 0:DH]
    d0 = acc_ref[0, :, DH:DH + 1]
    d1 = acc_ref[1, :, DH:DH + 1]
    msg = jnp.concatenate([m0 / (d0 + 1e-16), m1 / (d1 + 1e-16)], axis=1)
    o = jnp.dot(jax.nn.gelu(msg), wo_ref[...],
                preferred_element_type=jnp.float32) + misc_ref[0:1, :]
    xs = xs_ref[...]
    sig = misc_ref[1:2, :]
    out = sig * o + (1.0 - sig) * xs
    t = out + xs
    m = jnp.mean(t, axis=-1, keepdims=True)
    v = jnp.mean((t - m) * (t - m), axis=-1, keepdims=True)
    y_ref[...] = (t - m) / jnp.sqrt(v + 1e-5) * misc_ref[2:3, :] + misc_ref[3:4, :]


_post_call = pl.pallas_call(
    _post_body,
    grid=(GRID,),
    in_specs=[
        pl.BlockSpec((2, R, AW), lambda i: (0, i, 0)),
        pl.BlockSpec((R, C), lambda i: (i, 0)),
        pl.BlockSpec((C, C), lambda i: (0, 0)),
        pl.BlockSpec((4, C), lambda i: (0, 0)),
    ],
    out_specs=pl.BlockSpec((R, C), lambda i: (i, 0)),
    out_shape=jax.ShapeDtypeStruct((N, C), jnp.float32),
)


def _edge_body(qdA, kvA, srcA, dstA, qdB, kvB, srcB, dstB,
               z_hbm, outA, outB,
               isr0, isr1, idr0, idr1, iga0, iga1, igb0, igb1,
               isa0, isa1, isb0, isb1, sca0, sca1, scb0, scb1,
               qr0, qr1, kvr0, kvr1, wr0, wr1, acc,
               si0, si1, sq0, sq1, sk0, sk1, ss0, ss1):
    isr = (isr0, isr1)
    idr = (idr0, idr1)
    iga = (iga0, iga1)
    igb = (igb0, igb1)
    isa = (isa0, isa1)
    isb = (isb0, isb1)
    sca = (sca0, sca1)
    scb = (scb0, scb1)
    qr = (qr0, qr1)
    kvr = (kvr0, kvr1)
    wr = (wr0, wr1)
    si = (si0, si1)
    sq = (sq0, sq1)
    sk = (sk0, sk1)
    ss = (ss0, ss1)

    cid = lax.axis_index("c")
    sid = lax.axis_index("s")
    row0 = sid * ROWS_PER

    offv = jnp.broadcast_to(cid * N, (16,))
    lanes = lax.iota(jnp.int32, 16)
    zvec = jnp.zeros((16,), jnp.float32)
    junkv = jnp.full((16,), JUNK, jnp.int32)

    def run_phase(qd_hbm, kv_hbm, src_hbm, dst_hbm, out_hbm):
        def idx_start(m, b):
            ba = jnp.minimum(sid + m * NS, NCHUNK - 1) * B
            bb = jnp.minimum(ba + HB, E - HB)
            pltpu.async_copy(src_hbm.at[pl.ds(ba, HB)],
                             isr[b].at[pl.ds(0, HB)], si[b])
            pltpu.async_copy(src_hbm.at[pl.ds(bb, HB)],
                             isr[b].at[pl.ds(HB, HB)], si[b])
            pltpu.async_copy(dst_hbm.at[pl.ds(ba, HB)],
                             idr[b].at[pl.ds(0, HB)], si[b])
            pltpu.async_copy(dst_hbm.at[pl.ds(bb, HB)],
                             idr[b].at[pl.ds(HB, HB)], si[b])

        def idx_wait(b):
            for rf in (isr[b], idr[b]):
                for o in (0, HB):
                    pltpu.make_async_copy(src_hbm.at[pl.ds(0, HB)],
                                          rf.at[pl.ds(o, HB)], si[b]).wait()

        def addoff(b):
            for j in range(B // 16):
                sl = pl.ds(j * 16, 16)
                sl2 = pl.ds((j % 8) * 16, 16)
                tg = iga[b] if j < 8 else igb[b]
                ts = isa[b] if j < 8 else isb[b]
                tg[sl2] = idr[b][sl] + offv
                ts[sl2] = isr[b][sl] + offv

        def gather_start(b):
            pltpu.async_copy(qd_hbm.at[iga[b]], qr[b].at[pl.ds(0, HB)], sq[b])
            pltpu.async_copy(qd_hbm.at[igb[b]], qr[b].at[pl.ds(HB, HB)], sq[b])
            pltpu.async_copy(kv_hbm.at[isa[b]], kvr[b].at[pl.ds(0, HB)], sk[b])
            pltpu.async_copy(kv_hbm.at[isb[b]], kvr[b].at[pl.ds(HB, HB)], sk[b])

        def gather_wait(b):
            pltpu.make_async_copy(qd_hbm.at[iga[b]], qr[b].at[pl.ds(0, HB)], sq[b]).wait()
            pltpu.make_async_copy(qd_hbm.at[igb[b]], qr[b].at[pl.ds(HB, HB)], sq[b]).wait()
            pltpu.make_async_copy(kv_hbm.at[isa[b]], kvr[b].at[pl.ds(0, HB)], sk[b]).wait()
            pltpu.make_async_copy(kv_hbm.at[isb[b]], kvr[b].at[pl.ds(HB, HB)], sk[b]).wait()

        def scatter_wait(b):
            pltpu.make_async_copy(wr[b].at[pl.ds(0, HB)],
                                  acc.at[sca[b]], ss[b]).wait()
            pltpu.make_async_copy(wr[b].at[pl.ds(HB, HB)],
                                  acc.at[scb[b]], ss[b]).wait()

        idx_start(0, 0)
        idx_wait(0)
        addoff(0)
        gather_start(0)
        idx_start(1, 1)
        plsc.subcore_barrier()

        def iteration(n, p):
            q = 1 - p
            idx_wait(q)
            addoff(q)
            gather_start(q)

            @pl.when(n >= 2)
            def _():
                scatter_wait(p)

            e0 = (sid + n * NS) * B
            for j in range(B // 16):
                sl = pl.ds(j * 16, 16)
                sl2 = pl.ds((j % 8) * 16, 16)
                tgt = sca[p] if j < 8 else scb[p]
                egv = lanes + (e0 + j * 16)
                tgt[sl2] = jnp.where(egv < E, idr[p][sl], junkv)
            idx_start(n + 2, p)

            gather_wait(p)

            qrp, kvrp, wrp = qr[p], kvr[p], wr[p]

            @plsc.parallel_loop(0, B, unroll=4)
            def _(i):
                qa, qb = plsc.unpack(qrp[i, pl.ds(0, DH)],
                                     format=plsc.PackFormat.INTERLEAVED)
                ka, kb = plsc.unpack(kvrp[i, pl.ds(0, DH)],
                                     format=plsc.PackFormat.INTERLEAVED)
                h = qa * ka + qb * kb
                e = jnp.exp(jnp.broadcast_to(jnp.sum(h), (16,)))
                va, vb = plsc.unpack(kvrp[i, pl.ds(DH, DH)],
                                     format=plsc.PackFormat.INTERLEAVED)
                wb = vb * e
                wrp[i, pl.ds(0, 16)] = va * e
                wrp[i, pl.ds(16, 16)] = wb
                combo = jnp.where(lanes < 8, lax.rev(wb, (0,)),
                                  jnp.where(lanes < 9, e, zvec))
                wrp[i, pl.ds(24, 16)] = combo

            pltpu.async_copy(wr[p].at[pl.ds(0, HB)], acc.at[sca[p]],
                             ss[p], add=True)
            pltpu.async_copy(wr[p].at[pl.ds(HB, HB)], acc.at[scb[p]],
                             ss[p], add=True)

        def pair(m, carry):
            iteration(2 * m, 0)
            iteration(2 * m + 1, 1)
            return carry

        lax.fori_loop(0, NLOCP // 2, pair, 0)

        scatter_wait(0)
        scatter_wait(1)
        gather_wait(0)
        idx_wait(1)
        plsc.subcore_barrier()
        pltpu.sync_copy(acc.at[pl.ds(row0, ROWS_PER)],
                        out_hbm.at[cid, pl.ds(row0, ROWS_PER)])

    pltpu.sync_copy(z_hbm, acc.at[pl.ds(row0, ROWS_PER)])
    run_phase(qdA, kvA, srcA, dstA, outA)
    pltpu.sync_copy(z_hbm, acc.at[pl.ds(row0, ROWS_PER)])
    run_phase(qdB, kvB, srcB, dstB, outB)


@functools.cache
def _edge_call_factory():
    return pl.kernel(
        _edge_body,
        out_type=[jax.ShapeDtypeStruct((NC, NP, AW), jnp.float32)] * 2,
        mesh=plsc.VectorSubcoreMesh(core_axis_name="c", subcore_axis_name="s",
                                    num_cores=NC, num_subcores=NS),
        compiler_params=pltpu.CompilerParams(needs_layout_passes=False,
                                             use_tc_tiling_on_sc=False),
        scratch_types=(
            [pltpu.VMEM((B,), jnp.int32)] * 4
            + [pltpu.VMEM((HB,), jnp.int32)] * 12
            + [pltpu.VMEM((B, DH), jnp.bfloat16)] * 2
            + [pltpu.VMEM((B, 2 * DH), jnp.bfloat16)] * 2
            + [pltpu.VMEM((B, AW), jnp.float32)] * 2
            + [pltpu.VMEM_SHARED((NP, AW), jnp.float32)]
            + [pltpu.SemaphoreType.DMA] * 8
        ),
    )


def _blockdiag(a, scale):
    z = jnp.zeros((C, C), jnp.float32)
    z = z.at[0:DH, 0:DH].set(a[0] * scale[0])
    z = z.at[DH:C, DH:C].set(a[1] * scale[1])
    return z


def _fold(p, rel):
    s = rel["p"] / math.sqrt(DH)
    ablk = _blockdiag(rel["a"], s)
    mblk = _blockdiag(rel["m"], jnp.ones((H,), jnp.float32))
    wq, wk, wv = p["Wq"], p["Wk"] @ ablk, p["Wv"] @ mblk
    bq, bk, bv = p["bq"], p["bk"] @ ablk, p["bv"] @ mblk
    vq = jnp.asarray(_VQ)
    wcat = jnp.stack(
        [jnp.concatenate([wq[:, 0:DH], wk[:, 0:DH], wv[:, 0:DH][:, vq]], axis=1),
         jnp.concatenate([wq[:, DH:C], wk[:, DH:C], wv[:, DH:C][:, vq]], axis=1)])
    bcat = jnp.stack(
        [jnp.concatenate([bq[0:DH], bk[0:DH], bv[0:DH][vq]]),
         jnp.concatenate([bq[DH:C], bk[DH:C], bv[DH:C][vq]])])
    lnw = jnp.stack([p["b_in"], p["g_in"], p["b_ln_in"]])
    return wcat, bcat.reshape(H, 1, 3 * DH), lnw


def _misc(p):
    sig = jax.nn.sigmoid(p["skip"])
    return jnp.stack([p["bo"], jnp.full((C,), sig), p["g_out"], p["b_out"]])


def kernel(x_user, x_item, params, ei_user_rates_item, ei_item_rev_rates_user):
    pu, pi = params["user"], params["item"]
    ru, ri = params["rel"]["u2i"], params["rel"]["i2u"]

    wcat_u, bcat_u, lnw_u = _fold(pu, ru)
    wcat_i, bcat_i, lnw_i = _fold(pi, ri)

    xs_u, q_u, kv_u = _pre_call(x_user, pu["W_in"], lnw_u, wcat_u, bcat_u)
    xs_i, q_i, kv_i = _pre_call(x_item, pi["W_in"], lnw_i, wcat_i, bcat_i)

    z = jnp.zeros((ROWS_PER, AW), jnp.float32)
    src_ui = ei_user_rates_item[0].astype(jnp.int32)
    dst_ui = ei_user_rates_item[1].astype(jnp.int32)
    src_iu = ei_item_rev_rates_user[0].astype(jnp.int32)
    dst_iu = ei_item_rev_rates_user[1].astype(jnp.int32)

    edge = _edge_call_factory()
    acc_item, acc_user = edge(q_i, kv_u, src_ui, dst_ui,
                              q_u, kv_i, src_iu, dst_iu, z)

    y_user = _post_call(acc_user, xs_u, pu["Wo"], _misc(pu))
    y_item = _post_call(acc_item, xs_i, pi["Wo"], _misc(pi))
    return y_user, y_item
